# baseline (speedup 1.0000x reference)
import jax
import jax.numpy as jnp
from jax import lax
from jax.experimental import pallas as pl
from jax.experimental.pallas import tpu as pltpu
from jax.experimental.pallas import tpu_sc as plsc

NUM_CLASSES = 1000000
EMBED_DIM = 32
PAD_DIM = 128
BATCH = 16384
FIELDS = 26

_NC, _NS = 2, 16
_NW = _NC * _NS
_BW = BATCH // _NW
_SUB = 128
_NSUB = _BW // _SUB
_SKEW = PAD_DIM + 1
_L = 16


_NCHUNK = FIELDS * _NSUB
_NBUF = 4
_LAG = _NBUF - 1


def _embed_body(xt_hbm, xs_hbm, table_hbm, out_hbm, idx_v, idx2_v, rows, trans,
                sem_i, sems_g, sems_o):
    wid = lax.axis_index("s") * _NC + lax.axis_index("c")
    w4 = wid * _NSUB
    idx_cps = [
        pltpu.async_copy(xt_hbm.at[f, pl.ds(w4, _NSUB), :],
                         idx_v.at[pl.ds(f * _NSUB, _NSUB)], sem_i)
        for f in range(FIELDS)
    ] + [
        pltpu.async_copy(xs_hbm.at[f, pl.ds(w4, _NSUB), :],
                         idx2_v.at[pl.ds(f * _NSUB, _NSUB)], sem_i)
        for f in range(FIELDS)
    ]
    for cp in idx_cps:
        cp.wait()

    lane = lax.iota(jnp.int32, _L)
    row_consts = [rblk * _L + lane for rblk in range(_SUB // _L)]

    def issue_gather(g, b):
        return pltpu.async_copy(table_hbm.at[idx2_v.at[g]], rows[b],
                                sems_g[b])

    def wait_gather(g, b):
        pltpu.make_async_copy(table_hbm.at[idx2_v.at[g]], rows[b],
                              sems_g[b]).wait()

    def transpose_rows(g, src, dst):
        offs = [
            ((idx_v[g, pl.ds(rblk * _L, _L)] & 3) << 5)
            for rblk in range(_SUB // _L)
        ]

        def k_body(k, _):
            rot = (lane + k) & (_L - 1)
            for rblk in range(_SUB // _L):
                for e0 in range(0, EMBED_DIM, _L):
                    e_rows = e0 + rot
                    v = plsc.load_gather(src, [row_consts[rblk],
                                               offs[rblk] + e_rows])
                    plsc.store_scatter(dst, [e_rows, row_consts[rblk]], v)
            return ()
        lax.fori_loop(0, _L, k_body, ())

    def drain_out(b2):
        pltpu.make_async_copy(
            trans[b2],
            out_hbm.at[0, :, pl.ds(b2 * _SUB, _SUB)],
            sems_o[b2]).wait()

    def process(g, b):
        wait_gather(g, b)
        b2 = b & 1
        transpose_rows(g, rows[b], trans[b2])
        f = g // _NSUB
        s = g - f * _NSUB
        pltpu.async_copy(
            trans[b2],
            out_hbm.at[f, :, pl.ds(wid * _BW + s * _SUB, _SUB)],
            sems_o[b2])

    for b in range(_LAG):
        issue_gather(jnp.int32(b), b)

    def chunk_body(i, _):
        g0 = i * _NBUF
        for j in range(_NBUF):
            g = g0 + j
            issue_gather(g + _LAG, (j + _LAG) % _NBUF)
            if j >= 2:
                drain_out(j & 1)
            else:
                @pl.when(i > 0)
                def _drain_prev():
                    drain_out(j & 1)
            process(g, j)
        return ()

    n_full = (_NCHUNK - _LAG) // _NBUF
    lax.fori_loop(0, n_full, chunk_body, ())
    for g in range(n_full * _NBUF, _NCHUNK):
        b = g % _NBUF
        if g + _LAG < _NCHUNK:
            issue_gather(jnp.int32(g + _LAG), (g + _LAG) % _NBUF)
        drain_out(b & 1)
        process(jnp.int32(g), b)
    for b2 in range(2):
        drain_out(b2)


def kernel(x, table):
    mesh = plsc.VectorSubcoreMesh(core_axis_name="c", subcore_axis_name="s",
                                  num_cores=_NC, num_subcores=_NS)
    tp = table.reshape(NUM_CLASSES // 4, PAD_DIM)
    xt = x.T.reshape(FIELDS, BATCH // _SUB, _SUB)
    xs = (x >> 2).T.reshape(FIELDS, BATCH // _SUB, _SUB)
    out = pl.kernel(
        _embed_body,
        out_type=jax.ShapeDtypeStruct((FIELDS, EMBED_DIM, BATCH), jnp.float32),
        mesh=mesh,
        scratch_types=[
            pltpu.VMEM((_NCHUNK, _SUB), jnp.int32),
            pltpu.VMEM((_NCHUNK, _SUB), jnp.int32),
            [pltpu.VMEM((_SUB, PAD_DIM), jnp.float32)] * _NBUF,
            [pltpu.VMEM((EMBED_DIM, _SUB), jnp.float32)] * 2,
            pltpu.SemaphoreType.DMA,
            [pltpu.SemaphoreType.DMA] * _NBUF,
            [pltpu.SemaphoreType.DMA] * 2,
        ],
        compiler_params=pltpu.CompilerParams(use_tc_tiling_on_sc=True,
                                             needs_layout_passes=False),
    )(xt, xs, tp)
    return out.transpose(2, 0, 1)


# speedup vs baseline: 1.0008x; 1.0008x over previous
import jax
import jax.numpy as jnp
from jax import lax
from jax.experimental import pallas as pl
from jax.experimental.pallas import tpu as pltpu
from jax.experimental.pallas import tpu_sc as plsc

NUM_CLASSES = 1000000
EMBED_DIM = 32
PAD_DIM = 128
BATCH = 16384
FIELDS = 26

_NC, _NS = 2, 16
_NW = _NC * _NS
_BW = BATCH // _NW
_SUB = 128
_NSUB = _BW // _SUB
_L = 16


_NCHUNK = FIELDS * _NSUB
_NBUF = 4
_LAG = _NBUF - 1


def _embed_body(xt_hbm, xs_hbm, table_hbm, out_hbm, idx_v, idx2_v, rows, trans,
                sem_i, sems_g, sems_o):
    wid = lax.axis_index("s") * _NC + lax.axis_index("c")
    w4 = wid * _NSUB
    idx_cps = [
        pltpu.async_copy(xt_hbm.at[f, pl.ds(w4, _NSUB), :],
                         idx_v.at[pl.ds(f * _NSUB, _NSUB)], sem_i)
        for f in range(FIELDS)
    ] + [
        pltpu.async_copy(xs_hbm.at[f, pl.ds(w4, _NSUB), :],
                         idx2_v.at[pl.ds(f * _NSUB, _NSUB)], sem_i)
        for f in range(FIELDS)
    ]
    for cp in idx_cps:
        cp.wait()

    lane = lax.iota(jnp.int32, _L)
    row_consts = [rblk * _L + lane for rblk in range(_SUB // _L)]

    def issue_gather(g, b):
        return pltpu.async_copy(table_hbm.at[idx2_v.at[g]], rows[b],
                                sems_g[b])

    def wait_gather(g, b):
        pltpu.make_async_copy(table_hbm.at[idx2_v.at[g]], rows[b],
                              sems_g[b]).wait()

    def transpose_rows(g, src, dst):
        offs = [
            ((idx_v[g, pl.ds(rblk * _L, _L)] & 3) << 5)
            for rblk in range(_SUB // _L)
        ]

        def k_body(k, _):
            rot = (lane + k) & (_L - 1)
            for rblk in range(_SUB // _L):
                for e0 in range(0, EMBED_DIM, _L):
                    e_rows = e0 + rot
                    v = plsc.load_gather(src, [row_consts[rblk],
                                               offs[rblk] + e_rows])
                    plsc.store_scatter(dst, [e_rows, row_consts[rblk]], v)
            return ()
        lax.fori_loop(0, _L, k_body, ())

    def drain_out(b2):
        pltpu.make_async_copy(
            trans[b2],
            out_hbm.at[0, :, pl.ds(b2 * _SUB, _SUB)],
            sems_o[b2]).wait()

    def process(g, b):
        wait_gather(g, b)
        b2 = b & 1
        transpose_rows(g, rows[b], trans[b2])
        f = g // _NSUB
        s = g - f * _NSUB
        pltpu.async_copy(
            trans[b2],
            out_hbm.at[f, :, pl.ds(wid * _BW + s * _SUB, _SUB)],
            sems_o[b2])

    for b in range(_LAG):
        issue_gather(jnp.int32(b), b)

    def chunk_body(i, _):
        g0 = i * _NBUF
        for j in range(_NBUF):
            g = g0 + j
            issue_gather(g + _LAG, (j + _LAG) % _NBUF)
            if j >= 2:
                drain_out(j & 1)
            else:
                @pl.when(i > 0)
                def _drain_prev():
                    drain_out(j & 1)
            process(g, j)
        return ()

    n_full = (_NCHUNK - _LAG) // _NBUF
    lax.fori_loop(0, n_full, chunk_body, ())
    for g in range(n_full * _NBUF, _NCHUNK):
        b = g % _NBUF
        if g + _LAG < _NCHUNK:
            issue_gather(jnp.int32(g + _LAG), (g + _LAG) % _NBUF)
        drain_out(b & 1)
        process(jnp.int32(g), b)
    for b2 in range(2):
        drain_out(b2)


def kernel(x, table):
    mesh = plsc.VectorSubcoreMesh(core_axis_name="c", subcore_axis_name="s",
                                  num_cores=_NC, num_subcores=_NS)
    tp = table.reshape(NUM_CLASSES // 4, PAD_DIM)
    xt = x.T.reshape(FIELDS, BATCH // _SUB, _SUB)
    xs = (x >> 2).T.reshape(FIELDS, BATCH // _SUB, _SUB)
    out = pl.kernel(
        _embed_body,
        out_type=jax.ShapeDtypeStruct((FIELDS, EMBED_DIM, BATCH), jnp.float32),
        mesh=mesh,
        scratch_types=[
            pltpu.VMEM((_NCHUNK, _SUB), jnp.int32),
            pltpu.VMEM((_NCHUNK, _SUB), jnp.int32),
            [pltpu.VMEM((_SUB, PAD_DIM), jnp.float32)] * _NBUF,
            [pltpu.VMEM((EMBED_DIM, _SUB), jnp.float32)] * 2,
            pltpu.SemaphoreType.DMA,
            [pltpu.SemaphoreType.DMA] * _NBUF,
            [pltpu.SemaphoreType.DMA] * 2,
        ],
        compiler_params=pltpu.CompilerParams(use_tc_tiling_on_sc=True,
                                             needs_layout_passes=False),
    )(xt, xs, tp)
    return out.transpose(2, 0, 1)
